# baseline (speedup 1.0000x reference)
import functools

import jax
import jax.numpy as jnp
from jax.experimental import pallas as pl
from jax.experimental.pallas import tpu as pltpu

_NG = 256
_B = 20000
_MWIN = 104


def _accum_window(ids, ew_row, x, accx, accw, accc, base, mwin):
    b = ids.shape[1]
    rel = ids - base
    ri = jax.lax.broadcasted_iota(jnp.int32, (mwin, b), 0)
    eq = ri == rel
    ohw = jnp.where(eq, ew_row, 0.0)
    ctb = jnp.dot(ohw, x, preferred_element_type=jnp.float32)
    accx[pl.ds(base, mwin), :] += ctb
    accw[pl.ds(base, mwin), :] += jnp.sum(ohw, axis=1, keepdims=True)
    accc[pl.ds(base, mwin), :] += jnp.sum(eq.astype(jnp.float32), axis=1,
                                          keepdims=True)


def _pool_body(batch_ref, x_ref, W1t_ref, b1c_ref, W2t_ref, b2_ref, out_ref,
               accx, accw, accc, m_ref, *, n_total):
    i = pl.program_id(0)
    nb = pl.num_programs(0)

    @pl.when(i == 0)
    def _init():
        accx[...] = jnp.zeros_like(accx)
        accw[...] = jnp.zeros_like(accw)
        accc[...] = jnp.zeros_like(accc)
        m_ref[0, 0] = -jnp.inf

    x = x_ref[...]
    ht = jax.lax.dot_general(W1t_ref[...], x,
                             (((1,), (1,)), ((), ())),
                             preferred_element_type=jnp.float32)
    ht = jnp.maximum(ht + b1c_ref[...], 0.0)
    wt = jnp.dot(W2t_ref[...], ht,
                 preferred_element_type=jnp.float32) + b2_ref[0, 0]

    bm = jnp.max(wt)
    m_old = m_ref[0, 0]
    m_new = jnp.maximum(m_old, bm)
    scale = jnp.exp(m_old - m_new)
    m_ref[0, 0] = m_new

    ew_row = jnp.exp(wt - m_new)
    ids = batch_ref[0]

    accx[...] = accx[...] * scale
    accw[...] = accw[...] * scale

    g_min = jnp.min(ids)
    g_max = jnp.max(ids)
    fits = (g_max - g_min) <= (_MWIN - 9)
    base8 = jnp.minimum((g_min // 8) * 8, _NG - _MWIN)

    @pl.when(fits)
    def _fast():
        _accum_window(ids, ew_row, x, accx, accw, accc, base8, _MWIN)

    @pl.when(jnp.logical_not(fits))
    def _slow():
        _accum_window(ids, ew_row, x, accx, accw, accc, 0, _NG)

    @pl.when(i == nb - 1)
    def _fin():
        cnt = jnp.maximum(accc[...], 1.0)
        denom = (accw[...] / cnt) * float(n_total) + 1e-8
        out_ref[...] = accx[...] / (denom * cnt)


def _build_call(N, D, H, B, interpret=False):
    nb = N // B
    body = functools.partial(_pool_body, n_total=N)
    return pl.pallas_call(
        body,
        grid=(nb,),
        in_specs=[
            pl.BlockSpec((1, 1, B), lambda i: (i, 0, 0)),
            pl.BlockSpec((B, D), lambda i: (i, 0)),
            pl.BlockSpec((H, D), lambda i: (0, 0)),
            pl.BlockSpec((H, 1), lambda i: (0, 0)),
            pl.BlockSpec((1, H), lambda i: (0, 0)),
            pl.BlockSpec(memory_space=pltpu.SMEM),
        ],
        out_specs=pl.BlockSpec((_NG, D), lambda i: (0, 0)),
        out_shape=jax.ShapeDtypeStruct((_NG, D), jnp.float32),
        scratch_shapes=[
            pltpu.VMEM((_NG, D), jnp.float32),
            pltpu.VMEM((_NG, 1), jnp.float32),
            pltpu.VMEM((_NG, 1), jnp.float32),
            pltpu.SMEM((1, 1), jnp.float32),
        ],
        interpret=interpret,
    )


def kernel(x, batch, W1, b1, W2, b2):
    N, D = x.shape
    H = W1.shape[1]
    B = _B
    nb = N // B
    batch3d = batch.astype(jnp.int32).reshape(nb, 1, B)
    call = _build_call(N, D, H, B)
    return call(batch3d, x, W1.T, b1.reshape(H, 1), W2.T, b2.reshape(1, 1))


# speedup vs baseline: 1.1065x; 1.1065x over previous
import functools

import jax
import jax.numpy as jnp
from jax.experimental import pallas as pl
from jax.experimental.pallas import tpu as pltpu

_NG = 256
_B = 20000
_MWIN = 64


def _accum_window(ids, ew_row, x, accx, accw, accc, base, mwin):
    b = ids.shape[1]
    rel = ids - base
    ri = jax.lax.broadcasted_iota(jnp.int32, (mwin, b), 0)
    eq = ri == rel
    ohw = jnp.where(eq, ew_row, 0.0)
    ctb = jnp.dot(ohw, x, preferred_element_type=jnp.float32)
    accx[pl.ds(base, mwin), :] += ctb
    accw[pl.ds(base, mwin), :] += jnp.sum(ohw, axis=1, keepdims=True)
    accc[pl.ds(base, mwin), :] += jnp.sum(eq.astype(jnp.float32), axis=1,
                                          keepdims=True)


def _pool_body(batch_ref, x_ref, W1t_ref, b1c_ref, W2t_ref, b2_ref, out_ref,
               accx, accw, accc, m_ref, *, n_total):
    i = pl.program_id(0)
    nb = pl.num_programs(0)

    @pl.when(i == 0)
    def _init():
        accx[...] = jnp.zeros_like(accx)
        accw[...] = jnp.zeros_like(accw)
        accc[...] = jnp.zeros_like(accc)
        m_ref[0, 0] = -jnp.inf

    x = x_ref[...]
    ht = jax.lax.dot_general(W1t_ref[...], x,
                             (((1,), (1,)), ((), ())),
                             preferred_element_type=jnp.float32)
    ht = jnp.maximum(ht + b1c_ref[...], 0.0)
    wt = jnp.dot(W2t_ref[...], ht,
                 preferred_element_type=jnp.float32) + b2_ref[0, 0]

    bm = jnp.max(wt)
    m_old = m_ref[0, 0]
    m_new = jnp.maximum(m_old, bm)
    scale = jnp.exp(m_old - m_new)
    m_ref[0, 0] = m_new

    ew_row = jnp.exp(wt - m_new)
    ids = batch_ref[0]

    accx[...] = accx[...] * scale
    accw[...] = accw[...] * scale

    g_min = batch_ref[0, 0, 0]
    g_max = batch_ref[0, 0, ids.shape[1] - 1]
    fits = (g_max - g_min) <= (_MWIN - 9)
    base8 = jnp.minimum((g_min // 8) * 8, _NG - _MWIN)

    @pl.when(fits)
    def _fast():
        _accum_window(ids, ew_row, x, accx, accw, accc, base8, _MWIN)

    @pl.when(jnp.logical_not(fits))
    def _slow():
        _accum_window(ids, ew_row, x, accx, accw, accc, 0, _NG)

    @pl.when(i == nb - 1)
    def _fin():
        cnt = jnp.maximum(accc[...], 1.0)
        denom = (accw[...] / cnt) * float(n_total) + 1e-8
        out_ref[...] = accx[...] / (denom * cnt)


def _build_call(N, D, H, B, interpret=False):
    nb = N // B
    body = functools.partial(_pool_body, n_total=N)
    return pl.pallas_call(
        body,
        grid=(nb,),
        in_specs=[
            pl.BlockSpec((1, 1, B), lambda i: (i, 0, 0)),
            pl.BlockSpec((B, D), lambda i: (i, 0)),
            pl.BlockSpec((H, D), lambda i: (0, 0)),
            pl.BlockSpec((H, 1), lambda i: (0, 0)),
            pl.BlockSpec((1, H), lambda i: (0, 0)),
            pl.BlockSpec(memory_space=pltpu.SMEM),
        ],
        out_specs=pl.BlockSpec((_NG, D), lambda i: (0, 0)),
        out_shape=jax.ShapeDtypeStruct((_NG, D), jnp.float32),
        scratch_shapes=[
            pltpu.VMEM((_NG, D), jnp.float32),
            pltpu.VMEM((_NG, 1), jnp.float32),
            pltpu.VMEM((_NG, 1), jnp.float32),
            pltpu.SMEM((1, 1), jnp.float32),
        ],
        interpret=interpret,
    )


def kernel(x, batch, W1, b1, W2, b2):
    N, D = x.shape
    H = W1.shape[1]
    B = _B
    nb = N // B
    batch3d = batch.astype(jnp.int32).reshape(nb, 1, B)
    call = _build_call(N, D, H, B)
    return call(batch3d, x, W1.T, b1.reshape(H, 1), W2.T, b2.reshape(1, 1))
